# baseline (speedup 1.0000x reference)
import functools

import jax
import jax.numpy as jnp
from jax import lax
from jax.experimental import pallas as pl
from jax.experimental.pallas import tpu as pltpu
from jax.experimental.pallas import tpu_sc as plsc

N = 10000
E = 320000
D = 128
G = 16
NP = 10240
NC = 2
NS = 16
NW = NC * NS
CHR = 128
KR = 81
EE = NW * KR * CHR
CH = 64
KPT = 2 * KR
NBUF = 3
NR = KPT // NBUF
RPT = NP // NS

_mesh = plsc.VectorSubcoreMesh(
    core_axis_name="c", subcore_axis_name="s", num_cores=NC, num_subcores=NS)


@functools.partial(
    pl.kernel,
    out_type=jax.ShapeDtypeStruct((NC, NP), jnp.float32),
    mesh=_mesh,
    scratch_types=[
        pltpu.VMEM((KR, CHR), jnp.int32),
        pltpu.VMEM((KR, CHR), jnp.float32),
        pltpu.VMEM((RPT,), jnp.float32),
        pltpu.VMEM_SHARED((NP,), jnp.float32),
        pltpu.SemaphoreType.DMA,
    ],
)
def _deg_kernel(col_hbm, ew_hbm, out_hbm, col_v, ew_v, z_v, acc_sh, sem):
    cid = lax.axis_index("c")
    sid = lax.axis_index("s")
    wid = cid * NS + sid

    def _z(i, _):
        z_v[pl.ds(i * 16, 16)] = jnp.zeros((16,), jnp.float32)
        return 0
    lax.fori_loop(0, RPT // 16, _z, 0)
    pltpu.sync_copy(z_v, acc_sh.at[pl.ds(sid * RPT, RPT)])
    pltpu.sync_copy(col_hbm.at[wid], col_v)
    pltpu.sync_copy(ew_hbm.at[wid], ew_v)
    plsc.subcore_barrier()

    def _fire(k, _):
        pltpu.async_copy(ew_v.at[k], acc_sh.at[col_v.at[k]], sem, add=True)
        return 0
    lax.fori_loop(0, KR, _fire, 0)

    def _drain(k, _):
        pltpu.make_async_copy(ew_v.at[0], acc_sh.at[col_v.at[0]], sem).wait()
        return 0
    lax.fori_loop(0, KR, _drain, 0)
    plsc.subcore_barrier()
    pltpu.sync_copy(acc_sh.at[pl.ds(sid * RPT, RPT)],
                    out_hbm.at[cid, pl.ds(sid * RPT, RPT)])


@functools.partial(
    pl.kernel,
    out_type=jax.ShapeDtypeStruct((NW, KPT, CHR), jnp.int32),
    mesh=_mesh,
    compiler_params=pltpu.CompilerParams(needs_layout_passes=False),
    scratch_types=[
        pltpu.VMEM((NP,), jnp.float32),
        pltpu.VMEM((KR, CHR), jnp.int32),
        pltpu.VMEM((KR, CHR), jnp.float32),
        pltpu.VMEM((KPT, CHR), jnp.int32),
    ],
)
def _norm_kernel(dis_hbm, pk_hbm, ew_hbm, out_hbm,
                 dis_v, pk_v, ew_v, comb_v):
    wid = lax.axis_index("c") * NS + lax.axis_index("s")
    pltpu.sync_copy(dis_hbm, dis_v)
    pltpu.sync_copy(pk_hbm.at[wid], pk_v)
    pltpu.sync_copy(ew_hbm.at[wid], ew_v)

    def _body(k, _):
        for g in range(CHR // 16):
            s = pl.ds(g * 16, 16)
            pk16 = pk_v[k, s]
            dr = plsc.load_gather(dis_v, [pk16 & 0xFFFF])
            dc = plsc.load_gather(dis_v, [pk16 >> 16])
            nrm = dr * ew_v[k, s] * dc
            half = g // (CH // 16)
            go = (g % (CH // 16)) * 16
            comb_v[2 * k + half, pl.ds(go, 16)] = pk16
            comb_v[2 * k + half, pl.ds(CH + go, 16)] = plsc.bitcast(
                nrm, jnp.int32)
        return 0
    lax.fori_loop(0, KR, _body, 0)
    pltpu.sync_copy(comb_v, out_hbm.at[wid])


@functools.partial(
    pl.kernel,
    out_type=jax.ShapeDtypeStruct((NC, NP, D), jnp.float32),
    mesh=_mesh,
    compiler_params=pltpu.CompilerParams(needs_layout_passes=False),
    scratch_types=[
        pltpu.VMEM((KPT, CHR), jnp.int32),
        pltpu.VMEM((CH, D), jnp.float32),
        pltpu.VMEM((CH, D), jnp.float32),
        pltpu.VMEM((CH, D), jnp.float32),
        pltpu.VMEM((CH,), jnp.int32),
        pltpu.VMEM((CH,), jnp.int32),
        pltpu.VMEM((CH,), jnp.int32),
        pltpu.VMEM((CH,), jnp.int32),
        pltpu.VMEM((CH,), jnp.int32),
        pltpu.VMEM((CH,), jnp.int32),
        pltpu.VMEM_SHARED((NP, D), jnp.float32),
        pltpu.SemaphoreType.DMA,
        pltpu.SemaphoreType.DMA,
        pltpu.SemaphoreType.DMA,
        pltpu.SemaphoreType.DMA,
        pltpu.SemaphoreType.DMA,
        pltpu.SemaphoreType.DMA,
    ],
)
def _mp_kernel(tab_hbm, comb_hbm, zero_hbm, out_hbm,
               comb_v, rows0, rows1, rows2,
               ri0, ri1, ri2, ci0, ci1, ci2, acc_sh,
               g0, g1, g2, s0, s1, s2):
    cid = lax.axis_index("c")
    sid = lax.axis_index("s")
    wid = cid * NS + sid
    bufs = (rows0, rows1, rows2)
    ridx = (ri0, ri1, ri2)
    cidx = (ci0, ci1, ci2)
    gsems = (g0, g1, g2)
    ssems = (s0, s1, s2)

    pltpu.sync_copy(zero_hbm, acc_sh.at[pl.ds(sid * RPT, RPT)])
    pltpu.sync_copy(comb_hbm.at[wid], comb_v)
    plsc.subcore_barrier()

    def _stage_and_gather(k, b):
        for f in range(CH // 16):
            pk16 = comb_v[k, pl.ds(f * 16, 16)]
            sl = pl.ds(f * 16, 16)
            ridx[b][sl] = pk16 & 0xFFFF
            cidx[b][sl] = pk16 >> 16
        pltpu.async_copy(tab_hbm.at[ridx[b]], bufs[b], gsems[b])

    def _drain_scatter(b):
        pltpu.make_async_copy(bufs[b], acc_sh.at[cidx[b]],
                              ssems[b]).wait()

    def _process(k, b):
        pltpu.make_async_copy(tab_hbm.at[ridx[b]], bufs[b],
                              gsems[b]).wait()
        rows = bufs[b]
        for g in range(CH // 16):
            nv = plsc.bitcast(comb_v[k, pl.ds(CH + g * 16, 16)],
                              jnp.float32)
            for jj in range(16):
                s = nv[jj]
                for f in range(D // 16):
                    sl = pl.ds(f * 16, 16)
                    rows[g * 16 + jj, sl] = rows[g * 16 + jj, sl] * s
        pltpu.async_copy(rows, acc_sh.at[cidx[b]], ssems[b], add=True)

    _stage_and_gather(0, 0)

    def _round(r, _):
        for b in range(NBUF):
            k = r * NBUF + b
            b1 = (b + 1) % NBUF
            if b < NBUF - 1:
                @pl.when(r >= 1)
                def _():
                    _drain_scatter(b1)
                _stage_and_gather(k + 1, b1)
            else:
                _drain_scatter(b1)

                @pl.when(r < NR - 1)
                def _():
                    _stage_and_gather(k + 1, b1)
            _process(k, b)
        return 0
    lax.fori_loop(0, NR, _round, 0)
    _drain_scatter(1)
    _drain_scatter(2)
    plsc.subcore_barrier()
    pltpu.sync_copy(acc_sh.at[pl.ds(sid * RPT, RPT)],
                    out_hbm.at[cid, pl.ds(sid * RPT, RPT)])


BN = 2048


def _prep_body(p_ref, x_ref, w_ref, dis_ref, h_ref):
    deg = p_ref[0, :] + p_ref[1, :]
    dis_ref[...] = jnp.where(deg > 0, lax.rsqrt(deg), 0.0)
    h_ref[...] = jnp.dot(x_ref[...], w_ref[...],
                         preferred_element_type=jnp.float32)


def _prep(parts, x, w1):
    return pl.pallas_call(
        _prep_body,
        grid=(NP // BN,),
        in_specs=[
            pl.BlockSpec((NC, BN), lambda i: (0, i)),
            pl.BlockSpec((BN, D), lambda i: (i, 0)),
            pl.BlockSpec((D, D), lambda i: (0, 0)),
        ],
        out_specs=[
            pl.BlockSpec((BN,), lambda i: (i,)),
            pl.BlockSpec((BN, D), lambda i: (i, 0)),
        ],
        out_shape=[
            jax.ShapeDtypeStruct((NP,), jnp.float32),
            jax.ShapeDtypeStruct((NP, D), jnp.float32),
        ],
    )(parts, x, w1)


def _comb_body(p_ref, b_ref, w_ref, h_ref):
    h = p_ref[0] + p_ref[1] + b_ref[...][None, :]
    h = jnp.maximum(h, 0.0)
    h_ref[...] = jnp.dot(h, w_ref[...], preferred_element_type=jnp.float32)


def _combine(parts, b, w):
    return pl.pallas_call(
        _comb_body,
        grid=(NP // BN,),
        in_specs=[
            pl.BlockSpec((NC, BN, D), lambda i: (0, i, 0)),
            pl.BlockSpec((D,), lambda i: (0,)),
            pl.BlockSpec((D, D), lambda i: (0, 0)),
        ],
        out_specs=pl.BlockSpec((BN, D), lambda i: (i, 0)),
        out_shape=jax.ShapeDtypeStruct((NP, D), jnp.float32),
    )(parts, b, w)


def _head_body(p_ref, b3_ref, batch_ref, wm1_ref, bm1_ref, wm2_ref, bm2_ref,
               out_ref, pool_ref):
    i = pl.program_id(0)

    @pl.when(i == 0)
    def _():
        pool_ref[...] = jnp.zeros((G, D), jnp.float32)

    h = p_ref[0] + p_ref[1] + b3_ref[...][None, :]
    gid = lax.broadcasted_iota(jnp.int32, (G, BN), 0)
    mask = (batch_ref[...][None, :] == gid).astype(jnp.float32)
    pool_ref[...] += jnp.dot(mask, h, preferred_element_type=jnp.float32)

    @pl.when(i == NP // BN - 1)
    def _():
        z = jnp.dot(pool_ref[...], wm1_ref[...],
                    preferred_element_type=jnp.float32) + bm1_ref[...][None, :]
        z = jnp.maximum(z, 0.0)
        out_ref[...] = jnp.dot(z, wm2_ref[...],
                               preferred_element_type=jnp.float32) \
            + bm2_ref[...][None, :]


def _head(parts, b3, batch_ext, wm1, bm1, wm2, bm2):
    return pl.pallas_call(
        _head_body,
        grid=(NP // BN,),
        in_specs=[
            pl.BlockSpec((NC, BN, D), lambda i: (0, i, 0)),
            pl.BlockSpec((D,), lambda i: (0,)),
            pl.BlockSpec((BN,), lambda i: (i,)),
            pl.BlockSpec((D, D), lambda i: (0, 0)),
            pl.BlockSpec((D,), lambda i: (0,)),
            pl.BlockSpec((D, 64), lambda i: (0, 0)),
            pl.BlockSpec((64,), lambda i: (0,)),
        ],
        out_specs=pl.BlockSpec((G, 64), lambda i: (0, 0)),
        out_shape=jax.ShapeDtypeStruct((G, 64), jnp.float32),
        scratch_shapes=[pltpu.VMEM((G, D), jnp.float32)],
    )(parts, b3, batch_ext, wm1, bm1, wm2, bm2)


def kernel(x, edge_index, edge_attr, batch,
           W1, b1, W2, b2, W3, b3, Wm1, bm1, Wm2, bm2):
    row = edge_index[0]
    col = edge_index[1]
    ew = edge_attr[:, 0]

    loop = jnp.arange(N, dtype=jnp.int32)
    npad = EE - E - N
    row_e = jnp.concatenate([row, loop, jnp.zeros((npad,), jnp.int32)])
    col_e = jnp.concatenate([col, loop, jnp.zeros((npad,), jnp.int32)])
    ew_e = jnp.concatenate([ew, jnp.ones((N,), jnp.float32),
                            jnp.zeros((npad,), jnp.float32)])
    col2d = col_e.reshape(NW, KR, CHR)
    ew2d = ew_e.reshape(NW, KR, CHR)
    pk2d = (row_e | (col_e << 16)).reshape(NW, KR, CHR)

    x_p = jnp.concatenate([x, jnp.zeros((NP - N, D), jnp.float32)], axis=0)
    batch_ext = jnp.concatenate(
        [batch, jnp.full((NP - N,), G, jnp.int32)])
    zero_rows = jnp.zeros((RPT, D), jnp.float32)

    deg_parts = _deg_kernel(col2d, ew2d)
    dis, h = _prep(deg_parts, x_p, W1)
    comb2d = _norm_kernel(dis, pk2d, ew2d)

    parts = _mp_kernel(h, comb2d, zero_rows)
    h = _combine(parts, b1, W2)
    parts = _mp_kernel(h, comb2d, zero_rows)
    h = _combine(parts, b2, W3)
    parts = _mp_kernel(h, comb2d, zero_rows)
    return _head(parts, b3, batch_ext, Wm1, bm1, Wm2, bm2)


# speedup vs baseline: 1.1008x; 1.1008x over previous
import functools

import jax
import jax.numpy as jnp
from jax import lax
from jax.experimental import pallas as pl
from jax.experimental.pallas import tpu as pltpu
from jax.experimental.pallas import tpu_sc as plsc

N = 10000
E = 320000
D = 128
G = 16
NP = 10240
NC = 2
NS = 16
NW = NC * NS
CHR = 128
KA = 94
KB = 68
KRM = KA
EE = NW * 81 * CHR
EA = 16 * KA * CHR
CH = 64
NBUF = 2
RPT = NP // NS

_mesh = plsc.VectorSubcoreMesh(
    core_axis_name="c", subcore_axis_name="s", num_cores=NC, num_subcores=NS)


@functools.partial(
    pl.kernel,
    out_type=jax.ShapeDtypeStruct((NC, NP), jnp.float32),
    mesh=_mesh,
    scratch_types=[
        pltpu.VMEM((KRM, CHR), jnp.int32),
        pltpu.VMEM((KRM, CHR), jnp.float32),
        pltpu.VMEM((RPT,), jnp.float32),
        pltpu.VMEM_SHARED((NP,), jnp.float32),
        pltpu.SemaphoreType.DMA,
    ],
)
def _deg_kernel(col_hbm, ew_hbm, out_hbm, col_v, ew_v, z_v, acc_sh, sem):
    cid = lax.axis_index("c")
    sid = lax.axis_index("s")
    wid = cid * NS + sid
    kr = jnp.where(cid == 0, KA, KB)

    def _z(i, _):
        z_v[pl.ds(i * 16, 16)] = jnp.zeros((16,), jnp.float32)
        return 0
    lax.fori_loop(0, RPT // 16, _z, 0)
    pltpu.sync_copy(z_v, acc_sh.at[pl.ds(sid * RPT, RPT)])
    pltpu.sync_copy(col_hbm.at[wid], col_v)
    pltpu.sync_copy(ew_hbm.at[wid], ew_v)
    plsc.subcore_barrier()

    def _fire(k, _):
        pltpu.async_copy(ew_v.at[k], acc_sh.at[col_v.at[k]], sem, add=True)
        return 0
    lax.fori_loop(0, kr, _fire, 0)

    def _drain(k, _):
        pltpu.make_async_copy(ew_v.at[0], acc_sh.at[col_v.at[0]], sem).wait()
        return 0
    lax.fori_loop(0, kr, _drain, 0)
    plsc.subcore_barrier()
    pltpu.sync_copy(acc_sh.at[pl.ds(sid * RPT, RPT)],
                    out_hbm.at[cid, pl.ds(sid * RPT, RPT)])


@functools.partial(
    pl.kernel,
    out_type=jax.ShapeDtypeStruct((NW, KRM, CHR), jnp.float32),
    mesh=_mesh,
    compiler_params=pltpu.CompilerParams(needs_layout_passes=False),
    scratch_types=[
        pltpu.VMEM((NP,), jnp.float32),
        pltpu.VMEM((KRM, CHR), jnp.int32),
        pltpu.VMEM((KRM, CHR), jnp.int32),
        pltpu.VMEM((KRM, CHR), jnp.float32),
        pltpu.VMEM((KRM, CHR), jnp.float32),
    ],
)
def _norm_kernel(dis_hbm, row_hbm, col_hbm, ew_hbm, out_hbm,
                 dis_v, row_v, col_v, ew_v, nrm_v):
    cid = lax.axis_index("c")
    kr = jnp.where(cid == 0, KA, KB)
    wid = cid * NS + lax.axis_index("s")
    pltpu.sync_copy(dis_hbm, dis_v)
    pltpu.sync_copy(row_hbm.at[wid], row_v)
    pltpu.sync_copy(col_hbm.at[wid], col_v)
    pltpu.sync_copy(ew_hbm.at[wid], ew_v)

    def _body(k, _):
        for g in range(CHR // 16):
            s = pl.ds(g * 16, 16)
            dr = plsc.load_gather(dis_v, [row_v[k, s]])
            dc = plsc.load_gather(dis_v, [col_v[k, s]])
            nrm_v[k, s] = dr * ew_v[k, s] * dc
        return 0
    lax.fori_loop(0, kr, _body, 0)
    pltpu.sync_copy(nrm_v, out_hbm.at[wid])


@functools.partial(
    pl.kernel,
    out_type=jax.ShapeDtypeStruct((NC, NP, D), jnp.float32),
    mesh=_mesh,
    compiler_params=pltpu.CompilerParams(needs_layout_passes=False),
    scratch_types=[
        pltpu.VMEM((KRM, CHR), jnp.int32),
        pltpu.VMEM((KRM, CHR), jnp.float32),
        pltpu.VMEM((CH, D), jnp.float32),
        pltpu.VMEM((CH, D), jnp.float32),
        pltpu.VMEM((CH,), jnp.int32),
        pltpu.VMEM((CH,), jnp.int32),
        pltpu.VMEM((CH,), jnp.int32),
        pltpu.VMEM((CH,), jnp.int32),
        pltpu.VMEM_SHARED((NP, D), jnp.float32),
        pltpu.SemaphoreType.DMA,
        pltpu.SemaphoreType.DMA,
        pltpu.SemaphoreType.DMA,
        pltpu.SemaphoreType.DMA,
    ],
)
def _mp_kernel(tab_hbm, pk_hbm, nrm_hbm, zero_hbm, out_hbm,
               pk_v, nrm_v, rows0, rows1,
               ri0, ri1, ci0, ci1, acc_sh, g0, g1, s0, s1):
    cid = lax.axis_index("c")
    sid = lax.axis_index("s")
    wid = cid * NS + sid
    kr = jnp.where(cid == 0, KA, KB)
    bufs = (rows0, rows1)
    ridx = (ri0, ri1)
    cidx = (ci0, ci1)
    gsems = (g0, g1)
    ssems = (s0, s1)

    pltpu.sync_copy(zero_hbm, acc_sh.at[pl.ds(sid * RPT, RPT)])
    pltpu.sync_copy(pk_hbm.at[wid], pk_v)
    pltpu.sync_copy(nrm_hbm.at[wid], nrm_v)
    plsc.subcore_barrier()

    def _stage_and_gather(kk, b):
        for f in range(CH // 16):
            pk16 = pk_v[kk, pl.ds(b * CH + f * 16, 16)]
            sl = pl.ds(f * 16, 16)
            ridx[b][sl] = pk16 & 0xFFFF
            cidx[b][sl] = pk16 >> 16
        pltpu.async_copy(tab_hbm.at[ridx[b]], bufs[b], gsems[b])

    def _drain_scatter(b):
        pltpu.make_async_copy(bufs[b], acc_sh.at[cidx[b]],
                              ssems[b]).wait()

    def _process(kk, b):
        pltpu.make_async_copy(tab_hbm.at[ridx[b]], bufs[b],
                              gsems[b]).wait()
        rows = bufs[b]
        for g in range(CH // 16):
            nv = nrm_v[kk, pl.ds(b * CH + g * 16, 16)]
            for jj in range(16):
                s = nv[jj]
                for f in range(D // 16):
                    sl = pl.ds(f * 16, 16)
                    rows[g * 16 + jj, sl] = rows[g * 16 + jj, sl] * s
        pltpu.async_copy(rows, acc_sh.at[cidx[b]], ssems[b], add=True)

    _stage_and_gather(0, 0)

    def _round(r, _):
        @pl.when(r >= 1)
        def _():
            _drain_scatter(1)
        _stage_and_gather(r, 1)
        _process(r, 0)
        _drain_scatter(0)

        @pl.when(r < kr - 1)
        def _():
            _stage_and_gather(r + 1, 0)
        _process(r, 1)
        return 0
    lax.fori_loop(0, kr, _round, 0)
    _drain_scatter(1)
    plsc.subcore_barrier()
    pltpu.sync_copy(acc_sh.at[pl.ds(sid * RPT, RPT)],
                    out_hbm.at[cid, pl.ds(sid * RPT, RPT)])


BN = 2048


def _prep_body(p_ref, x_ref, w_ref, dis_ref, h_ref):
    deg = p_ref[0, :] + p_ref[1, :]
    dis_ref[...] = jnp.where(deg > 0, lax.rsqrt(deg), 0.0)
    h_ref[...] = jnp.dot(x_ref[...], w_ref[...],
                         preferred_element_type=jnp.float32)


def _prep(parts, x, w1):
    return pl.pallas_call(
        _prep_body,
        grid=(NP // BN,),
        in_specs=[
            pl.BlockSpec((NC, BN), lambda i: (0, i)),
            pl.BlockSpec((BN, D), lambda i: (i, 0)),
            pl.BlockSpec((D, D), lambda i: (0, 0)),
        ],
        out_specs=[
            pl.BlockSpec((BN,), lambda i: (i,)),
            pl.BlockSpec((BN, D), lambda i: (i, 0)),
        ],
        out_shape=[
            jax.ShapeDtypeStruct((NP,), jnp.float32),
            jax.ShapeDtypeStruct((NP, D), jnp.float32),
        ],
    )(parts, x, w1)


def _comb_body(p_ref, b_ref, w_ref, h_ref):
    h = p_ref[0] + p_ref[1] + b_ref[...][None, :]
    h = jnp.maximum(h, 0.0)
    h_ref[...] = jnp.dot(h, w_ref[...], preferred_element_type=jnp.float32)


def _combine(parts, b, w):
    return pl.pallas_call(
        _comb_body,
        grid=(NP // BN,),
        in_specs=[
            pl.BlockSpec((NC, BN, D), lambda i: (0, i, 0)),
            pl.BlockSpec((D,), lambda i: (0,)),
            pl.BlockSpec((D, D), lambda i: (0, 0)),
        ],
        out_specs=pl.BlockSpec((BN, D), lambda i: (i, 0)),
        out_shape=jax.ShapeDtypeStruct((NP, D), jnp.float32),
    )(parts, b, w)


def _head_body(p_ref, b3_ref, batch_ref, wm1_ref, bm1_ref, wm2_ref, bm2_ref,
               out_ref, pool_ref):
    i = pl.program_id(0)

    @pl.when(i == 0)
    def _():
        pool_ref[...] = jnp.zeros((G, D), jnp.float32)

    h = p_ref[0] + p_ref[1] + b3_ref[...][None, :]
    gid = lax.broadcasted_iota(jnp.int32, (G, BN), 0)
    mask = (batch_ref[...][None, :] == gid).astype(jnp.float32)
    pool_ref[...] += jnp.dot(mask, h, preferred_element_type=jnp.float32)

    @pl.when(i == NP // BN - 1)
    def _():
        z = jnp.dot(pool_ref[...], wm1_ref[...],
                    preferred_element_type=jnp.float32) + bm1_ref[...][None, :]
        z = jnp.maximum(z, 0.0)
        out_ref[...] = jnp.dot(z, wm2_ref[...],
                               preferred_element_type=jnp.float32) \
            + bm2_ref[...][None, :]


def _head(parts, b3, batch_ext, wm1, bm1, wm2, bm2):
    return pl.pallas_call(
        _head_body,
        grid=(NP // BN,),
        in_specs=[
            pl.BlockSpec((NC, BN, D), lambda i: (0, i, 0)),
            pl.BlockSpec((D,), lambda i: (0,)),
            pl.BlockSpec((BN,), lambda i: (i,)),
            pl.BlockSpec((D, D), lambda i: (0, 0)),
            pl.BlockSpec((D,), lambda i: (0,)),
            pl.BlockSpec((D, 64), lambda i: (0, 0)),
            pl.BlockSpec((64,), lambda i: (0,)),
        ],
        out_specs=pl.BlockSpec((G, 64), lambda i: (0, 0)),
        out_shape=jax.ShapeDtypeStruct((G, 64), jnp.float32),
        scratch_shapes=[pltpu.VMEM((G, D), jnp.float32)],
    )(parts, b3, batch_ext, wm1, bm1, wm2, bm2)


def kernel(x, edge_index, edge_attr, batch,
           W1, b1, W2, b2, W3, b3, Wm1, bm1, Wm2, bm2):
    row = edge_index[0]
    col = edge_index[1]
    ew = edge_attr[:, 0]

    loop = jnp.arange(N, dtype=jnp.int32)
    npad = EE - E - N
    row_e = jnp.concatenate([row, loop, jnp.zeros((npad,), jnp.int32)])
    col_e = jnp.concatenate([col, loop, jnp.zeros((npad,), jnp.int32)])
    ew_e = jnp.concatenate([ew, jnp.ones((N,), jnp.float32),
                            jnp.zeros((npad,), jnp.float32)])
    def _slab(v):
        a = v[:EA].reshape(NS, KA, CHR)
        b = v[EA:].reshape(NS, KB, CHR)
        b = jnp.pad(b, ((0, 0), (0, KA - KB), (0, 0)))
        return jnp.concatenate([a, b], axis=0)

    row2d = _slab(row_e)
    col2d = _slab(col_e)
    ew2d = _slab(ew_e)
    pk2d = _slab(row_e | (col_e << 16))

    x_p = jnp.concatenate([x, jnp.zeros((NP - N, D), jnp.float32)], axis=0)
    batch_ext = jnp.concatenate(
        [batch, jnp.full((NP - N,), G, jnp.int32)])
    zero_rows = jnp.zeros((RPT, D), jnp.float32)

    deg_parts = _deg_kernel(col2d, ew2d)
    dis, h = _prep(deg_parts, x_p, W1)
    norm2d = _norm_kernel(dis, row2d, col2d, ew2d)

    parts = _mp_kernel(h, pk2d, norm2d, zero_rows)
    h = _combine(parts, b1, W2)
    parts = _mp_kernel(h, pk2d, norm2d, zero_rows)
    h = _combine(parts, b2, W3)
    parts = _mp_kernel(h, pk2d, norm2d, zero_rows)
    return _head(parts, b3, batch_ext, Wm1, bm1, Wm2, bm2)
